# baseline (speedup 1.0000x reference)
import functools

import jax
import jax.numpy as jnp
from jax import lax
from jax.experimental import pallas as pl
from jax.experimental.pallas import tpu as pltpu
from jax.experimental.pallas import tpu_sc as plsc

B = 16384
D = 32
K = 20
U = K + 1
NC = 2
NS = 16
NW = NC * NS
BPW = B // NW
CH = 128
NCHUNK = BPW // CH
NGRP = CH // 16


def _sc_body(cpk_h, coff_h, pk_h, off_h, in_t, out_t,
             ctx_out, neg_out,
             cpk_v, coff_v, pk_v, off_v, crows, nbuf, ct, ctxaff_v, negaff_v,
             sem_c0, sem_c1, sem_n0, sem_n1):
    wid = lax.axis_index("s") * NC + lax.axis_index("c")
    base = wid * BPW

    pltpu.sync_copy(cpk_h.at[pl.ds(wid * NCHUNK, NCHUNK)], cpk_v)
    pltpu.sync_copy(coff_h.at[pl.ds(wid * NCHUNK, NCHUNK)], coff_v)
    pltpu.sync_copy(pk_h.at[:, pl.ds(wid * NCHUNK, NCHUNK), :], pk_v)
    pltpu.sync_copy(off_h.at[:, pl.ds(wid * NCHUNK, NCHUNK), :], off_v)

    iota16 = lax.iota(jnp.int32, 16)
    sem_c = (sem_c0, sem_c1)
    sem_n = (sem_n0, sem_n1)

    def fire_crows(c):
        return pltpu.async_copy(
            in_t.at[cpk_v.at[c]], crows.at[c % 2], sem_c[c % 2])

    def fire_unit(u, c, p):
        return pltpu.async_copy(
            out_t.at[pk_v.at[u, c]], nbuf.at[p], sem_n[p])

    def wait_unit(u, c, p):
        pltpu.make_async_copy(
            out_t.at[pk_v.at[u, c]], nbuf.at[p], sem_n[p]).wait()

    def dot16(u, c, p, g16):
        rows16 = g16 + iota16
        off = off_v[u, c, pl.ds(g16, 16)]
        acc = ct[0, pl.ds(g16, 16)] * plsc.load_gather(
            nbuf.at[p], [rows16, off])
        for d in range(1, D):
            acc = acc + ct[d, pl.ds(g16, 16)] * plsc.load_gather(
                nbuf.at[p], [rows16, off + d])
        return acc

    def compute_ctx(c, p):
        def group_body(g, _):
            g16 = g * 16
            ctxaff_v[pl.ds(c * CH + g16, 16)] = dot16(0, c, p, g16)
            return 0
        lax.fori_loop(0, NGRP, group_body, 0)

    def compute_neg(u, c, p):
        def group_body(g, _):
            g16 = g * 16
            negaff_v[u - 1, pl.ds(c * CH + g16, 16)] = dot16(u, c, p, g16)
            return 0
        lax.fori_loop(0, NGRP, group_body, 0)

    fire_crows(0)
    fire_unit(0, 0, 0)

    for c in range(NCHUNK):
        pc = c % 2
        pltpu.make_async_copy(
            in_t.at[cpk_v.at[c]], crows.at[c % 2], sem_c[c % 2]).wait()
        if c + 1 < NCHUNK:
            fire_crows(c + 1)

        def ct_body(g, _):
            g16 = g * 16
            rows16 = g16 + iota16
            coff = coff_v[c, pl.ds(g16, 16)]
            for d in range(D):
                ct[d, pl.ds(g16, 16)] = plsc.load_gather(
                    crows.at[c % 2], [rows16, coff + d])
            return 0

        lax.fori_loop(0, NGRP, ct_body, 0)

        fire_unit(1, c, 1 - pc)
        wait_unit(0, c, pc)
        compute_ctx(c, pc)

        def unit_pair(j, _):
            a = 1 + 2 * j
            fire_unit(a + 1, c, pc)
            wait_unit(a, c, 1 - pc)
            compute_neg(a, c, 1 - pc)
            fire_unit(a + 2, c, 1 - pc)
            wait_unit(a + 1, c, pc)
            compute_neg(a + 1, c, pc)
            return 0

        lax.fori_loop(0, (K - 2) // 2, unit_pair, 0)

        fire_unit(K, c, pc)
        wait_unit(K - 1, c, 1 - pc)
        compute_neg(K - 1, c, 1 - pc)
        if c + 1 < NCHUNK:
            fire_unit(0, c + 1, 1 - pc)
        wait_unit(K, c, pc)
        compute_neg(K, c, pc)

    pltpu.sync_copy(negaff_v, neg_out.at[:, pl.ds(base, BPW)])
    pltpu.sync_copy(ctxaff_v, ctx_out.at[pl.ds(base, BPW)])


_sc_affinities = functools.partial(
    pl.kernel,
    out_type=(
        jax.ShapeDtypeStruct((B,), jnp.float32),
        jax.ShapeDtypeStruct((K, B), jnp.float32),
    ),
    mesh=plsc.VectorSubcoreMesh(core_axis_name="c", subcore_axis_name="s"),
    compiler_params=pltpu.CompilerParams(
        needs_layout_passes=False, use_tc_tiling_on_sc=True),
    scratch_types=(
        pltpu.VMEM((NCHUNK, CH), jnp.int32),
        pltpu.VMEM((NCHUNK, CH), jnp.int32),
        pltpu.VMEM((U, NCHUNK, CH), jnp.int32),
        pltpu.VMEM((U, NCHUNK, CH), jnp.int32),
        pltpu.VMEM((2, CH, 128), jnp.float32),
        pltpu.VMEM((2, CH, 128), jnp.float32),
        pltpu.VMEM((D, CH), jnp.float32),
        pltpu.VMEM((BPW,), jnp.float32),
        pltpu.VMEM((K, BPW), jnp.float32),
        pltpu.SemaphoreType.DMA,
        pltpu.SemaphoreType.DMA,
        pltpu.SemaphoreType.DMA,
        pltpu.SemaphoreType.DMA,
    ),
)(_sc_body)


VBLK = 8192
Q = VBLK // 4
NTBLK = 123
RROWS = NTBLK * Q


def _repack_body(tin_ref, out_ref):
    x = tin_ref[...]
    acc = None
    for s in range(4):
        es = jnp.eye(D, 128, k=32 * s, dtype=jnp.float32)
        piece = jax.lax.dot_general(
            x[:, s * Q:(s + 1) * Q], es, (((0,), (0,)), ((), ())),
            precision=jax.lax.Precision.HIGHEST,
            preferred_element_type=jnp.float32)
        acc = piece if acc is None else acc + piece
    out_ref[...] = acc


def _repack(tT):
    return pl.pallas_call(
        _repack_body,
        grid=(NTBLK,),
        in_specs=[pl.BlockSpec((D, VBLK), lambda i: (0, i))],
        out_specs=pl.BlockSpec((Q, 128), lambda i: (i, 0)),
        out_shape=jax.ShapeDtypeStruct((RROWS, 128), jnp.float32),
    )(tT)


def _loss_body(ctx_ref, neg_ref, out_ref):
    ctx = ctx_ref[...]
    neg = -neg_ref[...]
    ls_c = jnp.minimum(ctx, 0.0) - jnp.log1p(jnp.exp(-jnp.abs(ctx)))
    ls_n = jnp.minimum(neg, 0.0) - jnp.log1p(jnp.exp(-jnp.abs(neg)))
    out_ref[0, 0] = -(jnp.sum(ls_c) / B) - (jnp.sum(ls_n) / (B * K))


def kernel(center, context, negatives, input_embedding, output_embedding):
    center = center.astype(jnp.int32)
    ctxneg = jnp.concatenate(
        [context.astype(jnp.int32)[None, :],
         negatives.astype(jnp.int32).T], axis=0)
    cpk = (((center >> 13) << 11) | (center & 2047)).reshape(B // CH, CH)
    coff = (((center >> 11) & 3) * D).reshape(B // CH, CH)
    pk = (((ctxneg >> 13) << 11) | (ctxneg & 2047)).reshape(U, B // CH, CH)
    off = (((ctxneg >> 11) & 3) * D).reshape(U, B // CH, CH)
    in_t = _repack(input_embedding.T)
    out_t = _repack(output_embedding.T)

    ctx_aff, neg_aff = _sc_affinities(cpk, coff, pk, off, in_t, out_t)

    loss = pl.pallas_call(
        _loss_body,
        out_shape=jax.ShapeDtypeStruct((1, 1), jnp.float32),
        out_specs=pl.BlockSpec(memory_space=pltpu.SMEM),
    )(ctx_aff.reshape(B // 128, 128), neg_aff.reshape(K * B // 128, 128))
    return loss[0, 0]


# speedup vs baseline: 1.6623x; 1.6623x over previous
import functools

import jax
import jax.numpy as jnp
from jax import lax
from jax.experimental import pallas as pl
from jax.experimental.pallas import tpu as pltpu
from jax.experimental.pallas import tpu_sc as plsc

B = 16384
D = 32
K = 20
U = K + 1
NC = 2
NS = 16
NW = NC * NS
BPW = B // NW
CH = 128
NCHUNK = BPW // CH
NGRP = CH // 16


def _sc_body(cpk_h, coff_h, pk_h, off_h, in_t, out_t,
             ctx_out, neg_out,
             cpk_v, coff_v, pk_v, off_v, crows, nbuf, ct, ctxaff_v, negaff_v,
             sem_c0, sem_c1, sem_n0, sem_n1):
    wid = lax.axis_index("s") * NC + lax.axis_index("c")
    base = wid * BPW

    pltpu.sync_copy(cpk_h.at[pl.ds(wid * NCHUNK, NCHUNK)], cpk_v)
    pltpu.sync_copy(coff_h.at[pl.ds(wid * NCHUNK, NCHUNK)], coff_v)
    pltpu.sync_copy(pk_h.at[:, pl.ds(wid * NCHUNK, NCHUNK), :], pk_v)
    pltpu.sync_copy(off_h.at[:, pl.ds(wid * NCHUNK, NCHUNK), :], off_v)

    iota16 = lax.iota(jnp.int32, 16)
    sem_c = (sem_c0, sem_c1)
    sem_n = (sem_n0, sem_n1)

    def fire_crows(c):
        return pltpu.async_copy(
            in_t.at[cpk_v.at[c]], crows.at[c % 2], sem_c[c % 2])

    def fire_unit(u, c, p):
        return pltpu.async_copy(
            out_t.at[pk_v.at[u, c]], nbuf.at[p], sem_n[p])

    def wait_unit(u, c, p):
        pltpu.make_async_copy(
            out_t.at[pk_v.at[u, c]], nbuf.at[p], sem_n[p]).wait()

    def dot16(u, c, p, g16):
        rows16 = g16 + iota16
        off = off_v[u, c, pl.ds(g16, 16)]
        acc = ct[0, pl.ds(g16, 16)] * plsc.load_gather(
            nbuf.at[p], [rows16, off])
        for d in range(1, D):
            acc = acc + ct[d, pl.ds(g16, 16)] * plsc.load_gather(
                nbuf.at[p], [rows16, off + d])
        return acc

    def compute_ctx(c, p):
        def group_body(g, _):
            g16 = g * 16
            ctxaff_v[pl.ds(c * CH + g16, 16)] = dot16(0, c, p, g16)
            return 0
        lax.fori_loop(0, NGRP, group_body, 0)

    def compute_neg(u, c, p):
        def group_body(g, _):
            g16 = g * 16
            negaff_v[u - 1, pl.ds(c * CH + g16, 16)] = dot16(u, c, p, g16)
            return 0
        lax.fori_loop(0, NGRP, group_body, 0)

    fire_crows(0)
    fire_unit(0, 0, 0)

    for c in range(NCHUNK):
        pc = c % 2
        pltpu.make_async_copy(
            in_t.at[cpk_v.at[c]], crows.at[c % 2], sem_c[c % 2]).wait()
        if c + 1 < NCHUNK:
            fire_crows(c + 1)

        def ct_body(g, _):
            g16 = g * 16
            rows16 = g16 + iota16
            coff = coff_v[c, pl.ds(g16, 16)]
            for d in range(D):
                ct[d, pl.ds(g16, 16)] = plsc.load_gather(
                    crows.at[c % 2], [rows16, coff + d])
            return 0

        lax.fori_loop(0, NGRP, ct_body, 0)

        fire_unit(1, c, 1 - pc)
        wait_unit(0, c, pc)
        compute_ctx(c, pc)

        def unit_pair(j, _):
            a = 1 + 2 * j
            fire_unit(a + 1, c, pc)
            wait_unit(a, c, 1 - pc)
            compute_neg(a, c, 1 - pc)
            fire_unit(a + 2, c, 1 - pc)
            wait_unit(a + 1, c, pc)
            compute_neg(a + 1, c, pc)
            return 0

        lax.fori_loop(0, (K - 2) // 2, unit_pair, 0)

        fire_unit(K, c, pc)
        wait_unit(K - 1, c, 1 - pc)
        compute_neg(K - 1, c, 1 - pc)
        if c + 1 < NCHUNK:
            fire_unit(0, c + 1, 1 - pc)
        wait_unit(K, c, pc)
        compute_neg(K, c, pc)

    pltpu.sync_copy(negaff_v, neg_out.at[:, pl.ds(base, BPW)])
    pltpu.sync_copy(ctxaff_v, ctx_out.at[pl.ds(base, BPW)])


_sc_affinities = functools.partial(
    pl.kernel,
    out_type=(
        jax.ShapeDtypeStruct((B,), jnp.float32),
        jax.ShapeDtypeStruct((K, B), jnp.float32),
    ),
    mesh=plsc.VectorSubcoreMesh(core_axis_name="c", subcore_axis_name="s"),
    compiler_params=pltpu.CompilerParams(
        needs_layout_passes=False, use_tc_tiling_on_sc=True),
    scratch_types=(
        pltpu.VMEM((NCHUNK, CH), jnp.int32),
        pltpu.VMEM((NCHUNK, CH), jnp.int32),
        pltpu.VMEM((U, NCHUNK, CH), jnp.int32),
        pltpu.VMEM((U, NCHUNK, CH), jnp.int32),
        pltpu.VMEM((2, CH, 128), jnp.float32),
        pltpu.VMEM((2, CH, 128), jnp.float32),
        pltpu.VMEM((D, CH), jnp.float32),
        pltpu.VMEM((BPW,), jnp.float32),
        pltpu.VMEM((K, BPW), jnp.float32),
        pltpu.SemaphoreType.DMA,
        pltpu.SemaphoreType.DMA,
        pltpu.SemaphoreType.DMA,
        pltpu.SemaphoreType.DMA,
    ),
)(_sc_body)


VBLK = 16384
Q = VBLK // 4
NTBLK = 62
RROWS = NTBLK * Q


def _repack_body(tin_ref, out_ref):
    x = tin_ref[...]
    pieces = [x[:, s * Q:(s + 1) * Q].T for s in range(4)]
    out_ref[...] = jnp.concatenate(pieces, axis=1)


def _repack(tT):
    return pl.pallas_call(
        _repack_body,
        grid=(NTBLK,),
        in_specs=[pl.BlockSpec((D, VBLK), lambda i: (0, i))],
        out_specs=pl.BlockSpec((Q, 128), lambda i: (i, 0)),
        out_shape=jax.ShapeDtypeStruct((RROWS, 128), jnp.float32),
    )(tT)


def _loss_body(ctx_ref, neg_ref, out_ref):
    ctx = ctx_ref[...]
    neg = -neg_ref[...]
    ls_c = jnp.minimum(ctx, 0.0) - jnp.log1p(jnp.exp(-jnp.abs(ctx)))
    ls_n = jnp.minimum(neg, 0.0) - jnp.log1p(jnp.exp(-jnp.abs(neg)))
    out_ref[0, 0] = -(jnp.sum(ls_c) / B) - (jnp.sum(ls_n) / (B * K))


def kernel(center, context, negatives, input_embedding, output_embedding):
    center = center.astype(jnp.int32)
    ctxneg = jnp.concatenate(
        [context.astype(jnp.int32)[None, :],
         negatives.astype(jnp.int32).T], axis=0)
    cpk = (((center >> 14) << 12) | (center & 4095)).reshape(B // CH, CH)
    coff = (((center >> 12) & 3) * D).reshape(B // CH, CH)
    pk = (((ctxneg >> 14) << 12) | (ctxneg & 4095)).reshape(U, B // CH, CH)
    off = (((ctxneg >> 12) & 3) * D).reshape(U, B // CH, CH)
    in_t = _repack(input_embedding.T)
    out_t = _repack(output_embedding.T)

    ctx_aff, neg_aff = _sc_affinities(cpk, coff, pk, off, in_t, out_t)

    loss = pl.pallas_call(
        _loss_body,
        out_shape=jax.ShapeDtypeStruct((1, 1), jnp.float32),
        out_specs=pl.BlockSpec(memory_space=pltpu.SMEM),
    )(ctx_aff.reshape(B // 128, 128), neg_aff.reshape(K * B // 128, 128))
    return loss[0, 0]


# speedup vs baseline: 1.6651x; 1.0017x over previous
import functools

import jax
import jax.numpy as jnp
from jax import lax
from jax.experimental import pallas as pl
from jax.experimental.pallas import tpu as pltpu
from jax.experimental.pallas import tpu_sc as plsc

B = 16384
D = 32
K = 20
U = K + 1
NC = 2
NS = 16
NW = NC * NS
BPW = B // NW
CH = 128
NCHUNK = BPW // CH
NGRP = CH // 16


def _sc_body(cpk_h, coff_h, pk_h, off_h, in_t, out_t,
             ctx_out, neg_out,
             cpk_v, coff_v, pk_v, off_v, crows, nbuf, ct, ctxaff_v, negaff_v,
             sem_c0, sem_c1, sem_n0, sem_n1, sem_n2):
    wid = lax.axis_index("s") * NC + lax.axis_index("c")
    base = wid * BPW

    pltpu.sync_copy(cpk_h.at[pl.ds(wid * NCHUNK, NCHUNK)], cpk_v)
    pltpu.sync_copy(coff_h.at[pl.ds(wid * NCHUNK, NCHUNK)], coff_v)
    pltpu.sync_copy(pk_h.at[:, pl.ds(wid * NCHUNK, NCHUNK), :], pk_v)
    pltpu.sync_copy(off_h.at[:, pl.ds(wid * NCHUNK, NCHUNK), :], off_v)

    iota16 = lax.iota(jnp.int32, 16)
    sem_c = (sem_c0, sem_c1)
    sem_n = (sem_n0, sem_n1, sem_n2)

    def fire_crows(c):
        return pltpu.async_copy(
            in_t.at[cpk_v.at[c]], crows.at[c % 2], sem_c[c % 2])

    def fire_unit(u, c, k):
        return pltpu.async_copy(
            out_t.at[pk_v.at[u, c]], nbuf.at[k], sem_n[k])

    def wait_unit(u, c, k):
        pltpu.make_async_copy(
            out_t.at[pk_v.at[u, c]], nbuf.at[k], sem_n[k]).wait()

    def dot16(u, c, k, g16):
        rows16 = g16 + iota16
        off = off_v[u, c, pl.ds(g16, 16)]
        acc = ct[0, pl.ds(g16, 16)] * plsc.load_gather(
            nbuf.at[k], [rows16, off])
        for d in range(1, D):
            acc = acc + ct[d, pl.ds(g16, 16)] * plsc.load_gather(
                nbuf.at[k], [rows16, off + d])
        return acc

    fire_crows(0)
    for k in range(3):
        fire_unit(k, 0, k)

    for c in range(NCHUNK):
        pltpu.make_async_copy(
            in_t.at[cpk_v.at[c]], crows.at[c % 2], sem_c[c % 2]).wait()
        if c + 1 < NCHUNK:
            fire_crows(c + 1)

        def ct_body(g, _):
            g16 = g * 16
            rows16 = g16 + iota16
            coff = coff_v[c, pl.ds(g16, 16)]
            for d in range(D):
                ct[d, pl.ds(g16, 16)] = plsc.load_gather(
                    crows.at[c % 2], [rows16, coff + d])
            return 0

        lax.fori_loop(0, NGRP, ct_body, 0)

        def triad(q, _):
            for k in range(3):
                u = 3 * q + k
                wait_unit(u, c, k)

                def group_body(g, _):
                    g16 = g * 16
                    acc = dot16(u, c, k, g16)

                    @pl.when(u == 0)
                    def _():
                        ctxaff_v[pl.ds(c * CH + g16, 16)] = acc

                    @pl.when(u > 0)
                    def _():
                        negaff_v[u - 1, pl.ds(c * CH + g16, 16)] = acc
                    return 0

                lax.fori_loop(0, NGRP, group_body, 0)

                if c + 1 < NCHUNK:
                    uf = jnp.where(q < 6, u + 3, k)
                    cf = jnp.where(q < 6, c, c + 1)
                    fire_unit(uf, cf, k)
                else:
                    @pl.when(q < 6)
                    def _():
                        fire_unit(u + 3, c, k)
            return 0

        lax.fori_loop(0, U // 3, triad, 0)

    pltpu.sync_copy(negaff_v, neg_out.at[:, pl.ds(base, BPW)])
    pltpu.sync_copy(ctxaff_v, ctx_out.at[pl.ds(base, BPW)])


_sc_affinities = functools.partial(
    pl.kernel,
    out_type=(
        jax.ShapeDtypeStruct((B,), jnp.float32),
        jax.ShapeDtypeStruct((K, B), jnp.float32),
    ),
    mesh=plsc.VectorSubcoreMesh(core_axis_name="c", subcore_axis_name="s"),
    compiler_params=pltpu.CompilerParams(
        needs_layout_passes=False, use_tc_tiling_on_sc=True),
    scratch_types=(
        pltpu.VMEM((NCHUNK, CH), jnp.int32),
        pltpu.VMEM((NCHUNK, CH), jnp.int32),
        pltpu.VMEM((U, NCHUNK, CH), jnp.int32),
        pltpu.VMEM((U, NCHUNK, CH), jnp.int32),
        pltpu.VMEM((2, CH, 128), jnp.float32),
        pltpu.VMEM((3, CH, 128), jnp.float32),
        pltpu.VMEM((D, CH), jnp.float32),
        pltpu.VMEM((BPW,), jnp.float32),
        pltpu.VMEM((K, BPW), jnp.float32),
        pltpu.SemaphoreType.DMA,
        pltpu.SemaphoreType.DMA,
        pltpu.SemaphoreType.DMA,
        pltpu.SemaphoreType.DMA,
        pltpu.SemaphoreType.DMA,
    ),
)(_sc_body)


VBLK = 16384
Q = VBLK // 4
NTBLK = 62
RROWS = NTBLK * Q


def _repack_body(tin_ref, out_ref):
    x = tin_ref[...]
    pieces = [x[:, s * Q:(s + 1) * Q].T for s in range(4)]
    out_ref[...] = jnp.concatenate(pieces, axis=1)


def _repack(tT):
    return pl.pallas_call(
        _repack_body,
        grid=(NTBLK,),
        in_specs=[pl.BlockSpec((D, VBLK), lambda i: (0, i))],
        out_specs=pl.BlockSpec((Q, 128), lambda i: (i, 0)),
        out_shape=jax.ShapeDtypeStruct((RROWS, 128), jnp.float32),
    )(tT)


def _loss_body(ctx_ref, neg_ref, out_ref):
    ctx = ctx_ref[...]
    neg = -neg_ref[...]
    ls_c = jnp.minimum(ctx, 0.0) - jnp.log1p(jnp.exp(-jnp.abs(ctx)))
    ls_n = jnp.minimum(neg, 0.0) - jnp.log1p(jnp.exp(-jnp.abs(neg)))
    out_ref[0, 0] = -(jnp.sum(ls_c) / B) - (jnp.sum(ls_n) / (B * K))


def kernel(center, context, negatives, input_embedding, output_embedding):
    center = center.astype(jnp.int32)
    ctxneg = jnp.concatenate(
        [context.astype(jnp.int32)[None, :],
         negatives.astype(jnp.int32).T], axis=0)
    cpk = (((center >> 14) << 12) | (center & 4095)).reshape(B // CH, CH)
    coff = (((center >> 12) & 3) * D).reshape(B // CH, CH)
    pk = (((ctxneg >> 14) << 12) | (ctxneg & 4095)).reshape(U, B // CH, CH)
    off = (((ctxneg >> 12) & 3) * D).reshape(U, B // CH, CH)
    in_t = _repack(input_embedding.T)
    out_t = _repack(output_embedding.T)

    ctx_aff, neg_aff = _sc_affinities(cpk, coff, pk, off, in_t, out_t)

    loss = pl.pallas_call(
        _loss_body,
        out_shape=jax.ShapeDtypeStruct((1, 1), jnp.float32),
        out_specs=pl.BlockSpec(memory_space=pltpu.SMEM),
    )(ctx_aff.reshape(B // 128, 128), neg_aff.reshape(K * B // 128, 128))
    return loss[0, 0]
